# baseline (speedup 1.0000x reference)
import functools

import jax
import jax.numpy as jnp
from jax import lax
from jax.experimental import pallas as pl
from jax.experimental.pallas import tpu as pltpu
from jax.experimental.pallas import tpu_sc as plsc


_LANE = 128
_G = 8
_NB = 2
_NW = 32
_GE = _G * _LANE


def _sc_gather(nodes_pad, e0, e1):
    rows_pad = e0.shape[0]
    rows_per_w = rows_pad // _NW
    groups = rows_per_w // _G
    main_iters = (groups - _NB) // _NB
    n_pad = rows_pad * _LANE

    mesh = plsc.VectorSubcoreMesh(core_axis_name="c", subcore_axis_name="s")
    out_sd = jax.ShapeDtypeStruct((8, n_pad), jnp.float32)

    scratch = (
        [pltpu.VMEM((_G, _LANE), jnp.int32) for _ in range(2 * _NB)]
        + [pltpu.VMEM((_GE, 8), jnp.float32) for _ in range(2 * _NB)]
        + [pltpu.VMEM((8, _GE), jnp.float32) for _ in range(2)]
        + [pltpu.VMEM_SHARED((nodes_pad.shape[0], 8), jnp.float32)]
        + [pltpu.SemaphoreType.DMA for _ in range(_NB)]
    )

    @functools.partial(
        pl.kernel,
        out_type=(out_sd, out_sd),
        mesh=mesh,
        scratch_types=scratch,
        compiler_params=pltpu.CompilerParams(
            use_tc_tiling_on_sc=False, needs_layout_passes=False),
    )
    def body(nodes_hbm, e0_ref, e1_ref, g1_ref, g2_ref, *scr):
        idx0 = scr[0:2 * _NB:2]
        idx1 = scr[1:2 * _NB:2]
        row0buf = scr[2 * _NB:4 * _NB:2]
        row1buf = scr[2 * _NB + 1:4 * _NB:2]
        t0, t1 = scr[4 * _NB:4 * _NB + 2]
        nodes_ref = scr[4 * _NB + 2]
        sems = scr[4 * _NB + 3:]
        sid = lax.axis_index("s")
        wid = sid * 2 + lax.axis_index("c")
        base_row = wid * rows_per_w

        @pl.when(sid == 0)
        def _load_table():
            pltpu.sync_copy(nodes_hbm, nodes_ref)

        plsc.subcore_barrier()
        iota16 = lax.iota(jnp.int32, 16)

        def start(g, b):
            r = base_row + g * _G
            pltpu.sync_copy(e0_ref.at[pl.ds(r, _G)], idx0[b])
            pltpu.sync_copy(e1_ref.at[pl.ds(r, _G)], idx1[b])
            for j in range(_G):
                pltpu.async_copy(nodes_ref.at[idx0[b].at[j]],
                                 row0buf[b].at[pl.ds(j * _LANE, _LANE)],
                                 sems[b])
                pltpu.async_copy(nodes_ref.at[idx1[b].at[j]],
                                 row1buf[b].at[pl.ds(j * _LANE, _LANE)],
                                 sems[b])

        def drain_store(g, b):
            for j in range(_G):
                pltpu.make_async_copy(
                    nodes_ref.at[idx0[b].at[j]],
                    row0buf[b].at[pl.ds(j * _LANE, _LANE)], sems[b]).wait()
                pltpu.make_async_copy(
                    nodes_ref.at[idx1[b].at[j]],
                    row1buf[b].at[pl.ds(j * _LANE, _LANE)], sems[b]).wait()

            def tbody(k, carry):
                ridx = k * 16 + iota16
                for c in range(8):
                    cidx = jnp.full((16,), c, jnp.int32)
                    t0[c, pl.ds(k * 16, 16)] = plsc.load_gather(
                        row0buf[b], [ridx, cidx])
                    t1[c, pl.ds(k * 16, 16)] = plsc.load_gather(
                        row1buf[b], [ridx, cidx])
                return carry

            lax.fori_loop(0, _GE // 16, tbody, 0)
            e_base = (base_row + g * _G) * _LANE
            pltpu.sync_copy(t0, g1_ref.at[:, pl.ds(e_base, _GE)])
            pltpu.sync_copy(t1, g2_ref.at[:, pl.ds(e_base, _GE)])

        for b in range(_NB):
            start(b, b)

        def loop_body(it, carry):
            for b in range(_NB):
                g = it * _NB + b
                drain_store(g, b)
                start(g + _NB, b)
            return carry

        lax.fori_loop(0, main_iters, loop_body, 0)

        for b in range(_NB):
            drain_store(main_iters * _NB + b, b)

    return body(nodes_pad, e0, e1)


def _cross(a, b):
    ax, ay, az = a
    bx, by, bz = b
    return (ay * bz - az * by, az * bx - ax * bz, ax * by - ay * bx)


def _qrot(q, v):
    qv = q[:3]
    qw = q[3]
    t = _cross(qv, v)
    t = (2.0 * t[0], 2.0 * t[1], 2.0 * t[2])
    u = _cross(qv, t)
    return (v[0] + qw * t[0] + u[0],
            v[1] + qw * t[1] + u[1],
            v[2] + qw * t[2] + u[2])


def _qmul(q, r):
    qx, qy, qz, qw = q
    rx, ry, rz, rw = r
    cx, cy, cz = _cross((qx, qy, qz), (rx, ry, rz))
    return (qw * rx + rw * qx + cx,
            qw * ry + rw * qy + cy,
            qw * rz + rw * qz + cz,
            qw * rw - (qx * rx + qy * ry + qz * rz))


def _se3_inv(t, q):
    qi = (-q[0], -q[1], -q[2], q[3])
    ti = _qrot(qi, t)
    return (-ti[0], -ti[1], -ti[2]), qi


def _se3_mul(ta, qa, tb, qb):
    r = _qrot(qa, tb)
    return (ta[0] + r[0], ta[1] + r[1], ta[2] + r[2]), _qmul(qa, qb)


def _tc_body(p_ref, a_ref, b_ref, o_ref):
    p = jnp.transpose(p_ref[...], (0, 2, 1))
    tp = (p[:, 0], p[:, 1], p[:, 2])
    qp = (p[:, 3], p[:, 4], p[:, 5], p[:, 6])
    t1 = (a_ref[0], a_ref[1], a_ref[2])
    q1 = (a_ref[3], a_ref[4], a_ref[5], a_ref[6])
    t2 = (b_ref[0], b_ref[1], b_ref[2])
    q2 = (b_ref[3], b_ref[4], b_ref[5], b_ref[6])

    tip, qip = _se3_inv(tp, qp)
    ti1, qi1 = _se3_inv(t1, q1)
    tc, qc = _se3_mul(tip, qip, ti1, qi1)
    te, qe = _se3_mul(tc, qc, t2, q2)

    sign = jnp.where(qe[3] < 0.0, -1.0, 1.0)
    qv = (qe[0] * sign, qe[1] * sign, qe[2] * sign)
    qw = qe[3] * sign
    n = jnp.sqrt(qv[0] * qv[0] + qv[1] * qv[1] + qv[2] * qv[2])
    theta = 2.0 * jnp.arctan2(n, qw)
    small = n < 1e-6
    n_safe = jnp.where(small, 1.0, n)
    factor = jnp.where(small, 2.0 / jnp.maximum(qw, 1e-6), theta / n_safe)
    phi = (factor * qv[0], factor * qv[1], factor * qv[2])

    th = jnp.sqrt(phi[0] * phi[0] + phi[1] * phi[1] + phi[2] * phi[2])
    small_t = th < 1e-6
    ths = jnp.where(small_t, 1.0, th)
    c = jnp.where(
        small_t,
        1.0 / 12.0,
        (1.0 - ths * jnp.sin(ths) / (2.0 * (1.0 - jnp.cos(ths))))
        / (ths * ths),
    )
    pt = _cross(phi, te)
    ppt = _cross(phi, pt)
    tau = (te[0] - 0.5 * pt[0] + c * ppt[0],
           te[1] - 0.5 * pt[1] + c * ppt[1],
           te[2] - 0.5 * pt[2] + c * ppt[2])
    rblk = o_ref.shape[0]
    out = jnp.concatenate(
        [x.reshape(rblk, 1, _LANE) for x in tau + phi], axis=1)
    o_ref[...] = jnp.transpose(out, (0, 2, 1))


def _tc_compute(pr, g1t, g2t, rblk):
    rows = pr.shape[0]
    grid = (rows // rblk,)
    return pl.pallas_call(
        _tc_body,
        grid=grid,
        in_specs=[
            pl.BlockSpec((rblk, _LANE, 7), lambda i: (i, 0, 0)),
            pl.BlockSpec((8, rblk, _LANE), lambda i: (0, i, 0)),
            pl.BlockSpec((8, rblk, _LANE), lambda i: (0, i, 0)),
        ],
        out_specs=pl.BlockSpec((rblk, _LANE, 6), lambda i: (i, 0, 0)),
        out_shape=jax.ShapeDtypeStruct((rows, _LANE, 6), jnp.float32),
    )(pr, g1t, g2t)


def kernel(nodes, edges, poses):
    n_edges = edges.shape[0]
    rows = n_edges // _LANE
    chunk = _NW * _G * _NB
    rows_pad = ((rows + chunk - 1) // chunk) * chunk

    nodes_pad = jnp.pad(nodes, ((0, 0), (0, 1)))
    et = edges.T.reshape(2, rows, _LANE)
    et = jnp.pad(et, ((0, 0), (0, rows_pad - rows), (0, 0)))

    n_pad = rows_pad * _LANE
    pr = jnp.pad(poses.reshape(rows, _LANE, 7),
                 ((0, rows_pad - rows), (0, 0), (0, 0)))

    split = (rows_pad // chunk // 2) * chunk
    outs = []
    for lo, hi in ((0, split), (split, rows_pad)):
        g1t, g2t = _sc_gather(nodes_pad, et[0, lo:hi], et[1, lo:hi])
        g1t = g1t.reshape(8, hi - lo, _LANE)
        g2t = g2t.reshape(8, hi - lo, _LANE)
        outs.append(_tc_compute(pr[lo:hi], g1t, g2t, 128))
    out = jnp.concatenate(outs, axis=0)
    return out.reshape(n_pad, 6)[:n_edges]


# speedup vs baseline: 1.2919x; 1.2919x over previous
import functools

import jax
import jax.numpy as jnp
from jax import lax
from jax.experimental import pallas as pl
from jax.experimental.pallas import tpu as pltpu
from jax.experimental.pallas import tpu_sc as plsc


_LANE = 128
_G = 8
_NB = 2
_NW = 32
_GE = _G * _LANE


def _sc_gather(nodes_pad, e0, e1):
    rows_pad = e0.shape[0]
    rows_per_w = rows_pad // _NW
    groups = rows_per_w // _G
    main_iters = (groups - _NB) // _NB
    n_pad = rows_pad * _LANE

    mesh = plsc.VectorSubcoreMesh(core_axis_name="c", subcore_axis_name="s")
    out_sd = jax.ShapeDtypeStruct((8, n_pad), jnp.float32)

    scratch = (
        [pltpu.VMEM((_G, _LANE), jnp.int32) for _ in range(2 * _NB)]
        + [pltpu.VMEM((_GE, 8), jnp.float32) for _ in range(2 * _NB)]
        + [pltpu.VMEM((8, _GE), jnp.float32) for _ in range(2)]
        + [pltpu.VMEM_SHARED((nodes_pad.shape[0], 8), jnp.float32)]
        + [pltpu.SemaphoreType.DMA for _ in range(_NB)]
    )

    @functools.partial(
        pl.kernel,
        out_type=(out_sd, out_sd),
        mesh=mesh,
        scratch_types=scratch,
        compiler_params=pltpu.CompilerParams(
            use_tc_tiling_on_sc=False, needs_layout_passes=False),
    )
    def body(nodes_hbm, e0_ref, e1_ref, g1_ref, g2_ref, *scr):
        idx0 = scr[0:2 * _NB:2]
        idx1 = scr[1:2 * _NB:2]
        row0buf = scr[2 * _NB:4 * _NB:2]
        row1buf = scr[2 * _NB + 1:4 * _NB:2]
        t0, t1 = scr[4 * _NB:4 * _NB + 2]
        nodes_ref = scr[4 * _NB + 2]
        sems = scr[4 * _NB + 3:]
        sid = lax.axis_index("s")
        wid = sid * 2 + lax.axis_index("c")
        base_row = wid * rows_per_w

        @pl.when(sid == 0)
        def _load_table():
            pltpu.sync_copy(nodes_hbm, nodes_ref)

        plsc.subcore_barrier()
        iota16 = lax.iota(jnp.int32, 16)

        def start(g, b):
            r = base_row + g * _G
            pltpu.sync_copy(e0_ref.at[pl.ds(r, _G)], idx0[b])
            pltpu.sync_copy(e1_ref.at[pl.ds(r, _G)], idx1[b])
            for j in range(_G):
                pltpu.async_copy(nodes_ref.at[idx0[b].at[j]],
                                 row0buf[b].at[pl.ds(j * _LANE, _LANE)],
                                 sems[b])
                pltpu.async_copy(nodes_ref.at[idx1[b].at[j]],
                                 row1buf[b].at[pl.ds(j * _LANE, _LANE)],
                                 sems[b])

        def drain_store(g, b):
            for j in range(_G):
                pltpu.make_async_copy(
                    nodes_ref.at[idx0[b].at[j]],
                    row0buf[b].at[pl.ds(j * _LANE, _LANE)], sems[b]).wait()
                pltpu.make_async_copy(
                    nodes_ref.at[idx1[b].at[j]],
                    row1buf[b].at[pl.ds(j * _LANE, _LANE)], sems[b]).wait()

            def tbody(k, carry):
                ridx = k * 16 + iota16
                for c in range(8):
                    cidx = jnp.full((16,), c, jnp.int32)
                    t0[c, pl.ds(k * 16, 16)] = plsc.load_gather(
                        row0buf[b], [ridx, cidx])
                    t1[c, pl.ds(k * 16, 16)] = plsc.load_gather(
                        row1buf[b], [ridx, cidx])
                return carry

            lax.fori_loop(0, _GE // 16, tbody, 0)
            e_base = (base_row + g * _G) * _LANE
            pltpu.sync_copy(t0, g1_ref.at[:, pl.ds(e_base, _GE)])
            pltpu.sync_copy(t1, g2_ref.at[:, pl.ds(e_base, _GE)])

        for b in range(_NB):
            start(b, b)

        def loop_body(it, carry):
            for b in range(_NB):
                g = it * _NB + b
                drain_store(g, b)
                start(g + _NB, b)
            return carry

        lax.fori_loop(0, main_iters, loop_body, 0)

        for b in range(_NB):
            drain_store(main_iters * _NB + b, b)

    return body(nodes_pad, e0, e1)


def _cross(a, b):
    ax, ay, az = a
    bx, by, bz = b
    return (ay * bz - az * by, az * bx - ax * bz, ax * by - ay * bx)


def _qrot(q, v):
    qv = q[:3]
    qw = q[3]
    t = _cross(qv, v)
    t = (2.0 * t[0], 2.0 * t[1], 2.0 * t[2])
    u = _cross(qv, t)
    return (v[0] + qw * t[0] + u[0],
            v[1] + qw * t[1] + u[1],
            v[2] + qw * t[2] + u[2])


def _qmul(q, r):
    qx, qy, qz, qw = q
    rx, ry, rz, rw = r
    cx, cy, cz = _cross((qx, qy, qz), (rx, ry, rz))
    return (qw * rx + rw * qx + cx,
            qw * ry + rw * qy + cy,
            qw * rz + rw * qz + cz,
            qw * rw - (qx * rx + qy * ry + qz * rz))


def _se3_inv(t, q):
    qi = (-q[0], -q[1], -q[2], q[3])
    ti = _qrot(qi, t)
    return (-ti[0], -ti[1], -ti[2]), qi


def _se3_mul(ta, qa, tb, qb):
    r = _qrot(qa, tb)
    return (ta[0] + r[0], ta[1] + r[1], ta[2] + r[2]), _qmul(qa, qb)


def _tc_body(p_ref, a_ref, b_ref, o_ref, ps_ref):
    p = jnp.transpose(p_ref[...], (0, 2, 1))
    for cdx in range(7):
        ps_ref[cdx] = p[:, cdx]
    tp = (ps_ref[0], ps_ref[1], ps_ref[2])
    qp = (ps_ref[3], ps_ref[4], ps_ref[5], ps_ref[6])
    t1 = (a_ref[0], a_ref[1], a_ref[2])
    q1 = (a_ref[3], a_ref[4], a_ref[5], a_ref[6])
    t2 = (b_ref[0], b_ref[1], b_ref[2])
    q2 = (b_ref[3], b_ref[4], b_ref[5], b_ref[6])

    tip, qip = _se3_inv(tp, qp)
    ti1, qi1 = _se3_inv(t1, q1)
    tc, qc = _se3_mul(tip, qip, ti1, qi1)
    te, qe = _se3_mul(tc, qc, t2, q2)

    sign = jnp.where(qe[3] < 0.0, -1.0, 1.0)
    qv = (qe[0] * sign, qe[1] * sign, qe[2] * sign)
    qw = qe[3] * sign
    n = jnp.sqrt(qv[0] * qv[0] + qv[1] * qv[1] + qv[2] * qv[2])
    theta = 2.0 * jnp.arctan2(n, qw)
    small = n < 1e-6
    n_safe = jnp.where(small, 1.0, n)
    factor = jnp.where(small, 2.0 / jnp.maximum(qw, 1e-6), theta / n_safe)
    phi = (factor * qv[0], factor * qv[1], factor * qv[2])

    th = jnp.sqrt(phi[0] * phi[0] + phi[1] * phi[1] + phi[2] * phi[2])
    small_t = th < 1e-6
    ths = jnp.where(small_t, 1.0, th)
    c = jnp.where(
        small_t,
        1.0 / 12.0,
        (1.0 - ths * jnp.sin(ths) / (2.0 * (1.0 - jnp.cos(ths))))
        / (ths * ths),
    )
    pt = _cross(phi, te)
    ppt = _cross(phi, pt)
    tau = (te[0] - 0.5 * pt[0] + c * ppt[0],
           te[1] - 0.5 * pt[1] + c * ppt[1],
           te[2] - 0.5 * pt[2] + c * ppt[2])
    rblk = o_ref.shape[0]
    out = jnp.concatenate(
        [x.reshape(rblk, 1, _LANE) for x in tau + phi], axis=1)
    o_ref[...] = jnp.transpose(out, (0, 2, 1))


def _tc_compute(pr, g1t, g2t, rblk):
    rows = pr.shape[0]
    grid = (rows // rblk,)
    return pl.pallas_call(
        _tc_body,
        grid=grid,
        in_specs=[
            pl.BlockSpec((rblk, _LANE, 7), lambda i: (i, 0, 0)),
            pl.BlockSpec((8, rblk, _LANE), lambda i: (0, i, 0)),
            pl.BlockSpec((8, rblk, _LANE), lambda i: (0, i, 0)),
        ],
        out_specs=pl.BlockSpec((rblk, _LANE, 6), lambda i: (i, 0, 0)),
        out_shape=jax.ShapeDtypeStruct((rows, _LANE, 6), jnp.float32),
        scratch_shapes=[pltpu.VMEM((7, rblk, _LANE), jnp.float32)],
    )(pr, g1t, g2t)


def kernel(nodes, edges, poses):
    n_edges = edges.shape[0]
    rows = n_edges // _LANE
    chunk = _NW * _G * _NB
    rows_pad = ((rows + chunk - 1) // chunk) * chunk

    nodes_pad = jnp.pad(nodes, ((0, 0), (0, 1)))
    et = edges.T.reshape(2, rows, _LANE)
    et = jnp.pad(et, ((0, 0), (0, rows_pad - rows), (0, 0)))

    n_pad = rows_pad * _LANE
    pr = jnp.pad(poses.reshape(rows, _LANE, 7),
                 ((0, rows_pad - rows), (0, 0), (0, 0)))

    split = (rows_pad // chunk // 2) * chunk
    outs = []
    for lo, hi in ((0, split), (split, rows_pad)):
        g1t, g2t = _sc_gather(nodes_pad, et[0, lo:hi], et[1, lo:hi])
        g1t = g1t.reshape(8, hi - lo, _LANE)
        g2t = g2t.reshape(8, hi - lo, _LANE)
        outs.append(_tc_compute(pr[lo:hi], g1t, g2t, 128))
    out = jnp.concatenate(outs, axis=0)
    return out.reshape(n_pad, 6)[:n_edges]


# speedup vs baseline: 2.5866x; 2.0022x over previous
import functools

import jax
import jax.numpy as jnp
from jax import lax
from jax.experimental import pallas as pl
from jax.experimental.pallas import tpu as pltpu
from jax.experimental.pallas import tpu_sc as plsc


_LANE = 128
_G = 8
_NB = 2
_NW = 32
_GE = _G * _LANE


def _sc_gather(nodes_pad, e0, e1):
    rows_c = e0.shape[0]
    rows_per_w = rows_c // _NW
    groups = rows_per_w // _G
    main_iters = (groups - _NB) // _NB
    n_c = rows_c * _LANE

    mesh = plsc.VectorSubcoreMesh(core_axis_name="c", subcore_axis_name="s")
    out_sd = jax.ShapeDtypeStruct((8, n_c), jnp.float32)

    scratch = (
        [pltpu.VMEM((_G, _LANE), jnp.int32) for _ in range(2 * _NB)]
        + [pltpu.VMEM((_GE, 8), jnp.float32) for _ in range(2 * _NB)]
        + [pltpu.VMEM((8, _GE), jnp.float32) for _ in range(2)]
        + [pltpu.VMEM_SHARED((nodes_pad.shape[0], 8), jnp.float32)]
        + [pltpu.SemaphoreType.DMA for _ in range(2 * _NB)]
    )

    @functools.partial(
        pl.kernel,
        out_type=(out_sd, out_sd),
        mesh=mesh,
        scratch_types=scratch,
        compiler_params=pltpu.CompilerParams(
            use_tc_tiling_on_sc=False, needs_layout_passes=False),
    )
    def body(nodes_hbm, e0_ref, e1_ref, g1_ref, g2_ref, *scr):
        idx0 = scr[0:2 * _NB:2]
        idx1 = scr[1:2 * _NB:2]
        row0buf = scr[2 * _NB:4 * _NB:2]
        row1buf = scr[2 * _NB + 1:4 * _NB:2]
        t0, t1 = scr[4 * _NB:4 * _NB + 2]
        nodes_ref = scr[4 * _NB + 2]
        sems = scr[4 * _NB + 3:4 * _NB + 3 + _NB]
        sems_h = scr[4 * _NB + 3 + _NB:]
        sid = lax.axis_index("s")
        wid = sid * 2 + lax.axis_index("c")
        base_row = wid * rows_per_w

        @pl.when(sid == 0)
        def _load_table():
            pltpu.sync_copy(nodes_hbm, nodes_ref)

        plsc.subcore_barrier()
        iota16 = lax.iota(jnp.int32, 16)

        def start(g, b):
            r = base_row + g * _G
            pltpu.sync_copy(e0_ref.at[pl.ds(r, _G)], idx0[b])
            pltpu.sync_copy(e1_ref.at[pl.ds(r, _G)], idx1[b])
            for j in range(_G):
                pltpu.async_copy(nodes_ref.at[idx0[b].at[j]],
                                 row0buf[b].at[pl.ds(j * _LANE, _LANE)],
                                 sems[b])
                pltpu.async_copy(nodes_hbm.at[idx1[b].at[j]],
                                 row1buf[b].at[pl.ds(j * _LANE, _LANE)],
                                 sems_h[b])

        def drain_store(g, b):
            for j in range(_G):
                pltpu.make_async_copy(
                    nodes_ref.at[idx0[b].at[j]],
                    row0buf[b].at[pl.ds(j * _LANE, _LANE)], sems[b]).wait()
                pltpu.make_async_copy(
                    nodes_hbm.at[idx1[b].at[j]],
                    row1buf[b].at[pl.ds(j * _LANE, _LANE)], sems_h[b]).wait()

            def tbody(k, carry):
                ridx = k * 16 + iota16
                for c in range(8):
                    cidx = jnp.full((16,), c, jnp.int32)
                    t0[c, pl.ds(k * 16, 16)] = plsc.load_gather(
                        row0buf[b], [ridx, cidx])
                    t1[c, pl.ds(k * 16, 16)] = plsc.load_gather(
                        row1buf[b], [ridx, cidx])
                return carry

            lax.fori_loop(0, _GE // 16, tbody, 0)
            e_base = (base_row + g * _G) * _LANE
            pltpu.sync_copy(t0, g1_ref.at[:, pl.ds(e_base, _GE)])
            pltpu.sync_copy(t1, g2_ref.at[:, pl.ds(e_base, _GE)])

        for b in range(_NB):
            start(b, b)

        def loop_body(it, carry):
            for b in range(_NB):
                g = it * _NB + b
                drain_store(g, b)
                start(g + _NB, b)
            return carry

        lax.fori_loop(0, main_iters, loop_body, 0)

        for b in range(_NB):
            drain_store(main_iters * _NB + b, b)

    return body(nodes_pad, e0, e1)


def _cross(a, b):
    ax, ay, az = a
    bx, by, bz = b
    return (ay * bz - az * by, az * bx - ax * bz, ax * by - ay * bx)


def _qrot(q, v):
    qv = q[:3]
    qw = q[3]
    t = _cross(qv, v)
    t = (2.0 * t[0], 2.0 * t[1], 2.0 * t[2])
    u = _cross(qv, t)
    return (v[0] + qw * t[0] + u[0],
            v[1] + qw * t[1] + u[1],
            v[2] + qw * t[2] + u[2])


def _qmul(q, r):
    qx, qy, qz, qw = q
    rx, ry, rz, rw = r
    cx, cy, cz = _cross((qx, qy, qz), (rx, ry, rz))
    return (qw * rx + rw * qx + cx,
            qw * ry + rw * qy + cy,
            qw * rz + rw * qz + cz,
            qw * rw - (qx * rx + qy * ry + qz * rz))


def _se3_inv(t, q):
    qi = (-q[0], -q[1], -q[2], q[3])
    ti = _qrot(qi, t)
    return (-ti[0], -ti[1], -ti[2]), qi


def _se3_mul(ta, qa, tb, qb):
    r = _qrot(qa, tb)
    return (ta[0] + r[0], ta[1] + r[1], ta[2] + r[2]), _qmul(qa, qb)


def _tc_body(p_ref, a_ref, b_ref, o_ref):
    tp = (p_ref[0], p_ref[1], p_ref[2])
    qp = (p_ref[3], p_ref[4], p_ref[5], p_ref[6])
    t1 = (a_ref[0], a_ref[1], a_ref[2])
    q1 = (a_ref[3], a_ref[4], a_ref[5], a_ref[6])
    t2 = (b_ref[0], b_ref[1], b_ref[2])
    q2 = (b_ref[3], b_ref[4], b_ref[5], b_ref[6])

    tip, qip = _se3_inv(tp, qp)
    ti1, qi1 = _se3_inv(t1, q1)
    tc, qc = _se3_mul(tip, qip, ti1, qi1)
    te, qe = _se3_mul(tc, qc, t2, q2)

    sign = jnp.where(qe[3] < 0.0, -1.0, 1.0)
    qv = (qe[0] * sign, qe[1] * sign, qe[2] * sign)
    qw = qe[3] * sign
    n = jnp.sqrt(qv[0] * qv[0] + qv[1] * qv[1] + qv[2] * qv[2])
    theta = 2.0 * jnp.arctan2(n, qw)
    small = n < 1e-6
    n_safe = jnp.where(small, 1.0, n)
    factor = jnp.where(small, 2.0 / jnp.maximum(qw, 1e-6), theta / n_safe)
    phi = (factor * qv[0], factor * qv[1], factor * qv[2])

    th = jnp.sqrt(phi[0] * phi[0] + phi[1] * phi[1] + phi[2] * phi[2])
    small_t = th < 1e-6
    ths = jnp.where(small_t, 1.0, th)
    c = jnp.where(
        small_t,
        1.0 / 12.0,
        (1.0 - ths * jnp.sin(ths) / (2.0 * (1.0 - jnp.cos(ths))))
        / (ths * ths),
    )
    pt = _cross(phi, te)
    ppt = _cross(phi, pt)
    o_ref[0] = te[0] - 0.5 * pt[0] + c * ppt[0]
    o_ref[1] = te[1] - 0.5 * pt[1] + c * ppt[1]
    o_ref[2] = te[2] - 0.5 * pt[2] + c * ppt[2]
    o_ref[3] = phi[0]
    o_ref[4] = phi[1]
    o_ref[5] = phi[2]


def _tc_compute(pt, g1t, g2t, rblk):
    rows = pt.shape[1]
    grid = (rows // rblk,)
    return pl.pallas_call(
        _tc_body,
        grid=grid,
        in_specs=[
            pl.BlockSpec((7, rblk, _LANE), lambda i: (0, i, 0)),
            pl.BlockSpec((8, rblk, _LANE), lambda i: (0, i, 0)),
            pl.BlockSpec((8, rblk, _LANE), lambda i: (0, i, 0)),
        ],
        out_specs=pl.BlockSpec((6, rblk, _LANE), lambda i: (0, i, 0)),
        out_shape=jax.ShapeDtypeStruct((6, rows, _LANE), jnp.float32),
    )(pt, g1t, g2t)


def kernel(nodes, edges, poses):
    n_edges = edges.shape[0]
    rows = n_edges // _LANE
    chunk = _NW * _G * _NB
    rows_pad = ((rows + chunk - 1) // chunk) * chunk

    nodes_pad = jnp.pad(nodes, ((0, 0), (0, 1)))
    et = edges.T.reshape(2, rows, _LANE)
    et = jnp.pad(et, ((0, 0), (0, rows_pad - rows), (0, 0)))

    n_pad = rows_pad * _LANE
    pt = jnp.pad(poses.T, ((0, 0), (0, n_pad - n_edges)))
    pt = pt.reshape(7, rows_pad, _LANE)

    split = (rows_pad // chunk // 2) * chunk
    outs = []
    for lo, hi in ((0, split), (split, rows_pad)):
        g1t, g2t = _sc_gather(nodes_pad, et[0, lo:hi], et[1, lo:hi])
        g1t = g1t.reshape(8, hi - lo, _LANE)
        g2t = g2t.reshape(8, hi - lo, _LANE)
        outs.append(_tc_compute(pt[:, lo:hi], g1t, g2t, 128))
    out_t = jnp.concatenate(outs, axis=1)
    return out_t.reshape(6, n_pad)[:, :n_edges].T


# speedup vs baseline: 2.8436x; 1.0994x over previous
import functools

import jax
import jax.numpy as jnp
from jax import lax
from jax.experimental import pallas as pl
from jax.experimental.pallas import tpu as pltpu
from jax.experimental.pallas import tpu_sc as plsc


_LANE = 128
_G = 8
_NB = 2
_NW = 32
_GE = _G * _LANE


def _sc_gather(nodes_pad, e0, e1):
    rows_c = e0.shape[0]
    rows_per_w = rows_c // _NW
    groups = rows_per_w // _G
    main_iters = (groups - _NB) // _NB
    n_c = rows_c * _LANE

    mesh = plsc.VectorSubcoreMesh(core_axis_name="c", subcore_axis_name="s")
    out_sd = jax.ShapeDtypeStruct((8, n_c), jnp.float32)

    scratch = (
        [pltpu.VMEM((_G, _LANE), jnp.int32) for _ in range(2 * _NB)]
        + [pltpu.VMEM((_GE, 8), jnp.float32) for _ in range(2 * _NB)]
        + [pltpu.VMEM((8, _GE), jnp.float32) for _ in range(2)]
        + [pltpu.VMEM_SHARED((nodes_pad.shape[0], 8), jnp.float32)]
        + [pltpu.SemaphoreType.DMA for _ in range(2 * _NB)]
    )

    @functools.partial(
        pl.kernel,
        out_type=(out_sd, out_sd),
        mesh=mesh,
        scratch_types=scratch,
        compiler_params=pltpu.CompilerParams(
            use_tc_tiling_on_sc=False, needs_layout_passes=False),
    )
    def body(nodes_hbm, e0_ref, e1_ref, g1_ref, g2_ref, *scr):
        idx0 = scr[0:2 * _NB:2]
        idx1 = scr[1:2 * _NB:2]
        row0buf = scr[2 * _NB:4 * _NB:2]
        row1buf = scr[2 * _NB + 1:4 * _NB:2]
        t0, t1 = scr[4 * _NB:4 * _NB + 2]
        nodes_ref = scr[4 * _NB + 2]
        sems = scr[4 * _NB + 3:4 * _NB + 3 + _NB]
        sems_h = scr[4 * _NB + 3 + _NB:]
        sid = lax.axis_index("s")
        wid = sid * 2 + lax.axis_index("c")
        base_row = wid * rows_per_w

        @pl.when(sid == 0)
        def _load_table():
            pltpu.sync_copy(nodes_hbm, nodes_ref)

        plsc.subcore_barrier()
        iota16 = lax.iota(jnp.int32, 16)

        def start(g, b):
            r = base_row + g * _G
            pltpu.sync_copy(e0_ref.at[pl.ds(r, _G)], idx0[b])
            pltpu.sync_copy(e1_ref.at[pl.ds(r, _G)], idx1[b])
            for j in range(_G):
                pltpu.async_copy(nodes_ref.at[idx0[b].at[j]],
                                 row0buf[b].at[pl.ds(j * _LANE, _LANE)],
                                 sems[b])
                pltpu.async_copy(nodes_ref.at[idx1[b].at[j]],
                                 row1buf[b].at[pl.ds(j * _LANE, _LANE)],
                                 sems_h[b])

        def drain_store(g, b):
            for j in range(_G):
                pltpu.make_async_copy(
                    nodes_ref.at[idx0[b].at[j]],
                    row0buf[b].at[pl.ds(j * _LANE, _LANE)], sems[b]).wait()
                pltpu.make_async_copy(
                    nodes_ref.at[idx1[b].at[j]],
                    row1buf[b].at[pl.ds(j * _LANE, _LANE)], sems_h[b]).wait()

            def tbody(k, carry):
                ridx = k * 16 + iota16
                for c in range(8):
                    cidx = jnp.full((16,), c, jnp.int32)
                    t0[c, pl.ds(k * 16, 16)] = plsc.load_gather(
                        row0buf[b], [ridx, cidx])
                    t1[c, pl.ds(k * 16, 16)] = plsc.load_gather(
                        row1buf[b], [ridx, cidx])
                return carry

            lax.fori_loop(0, _GE // 16, tbody, 0)
            e_base = (base_row + g * _G) * _LANE
            pltpu.sync_copy(t0, g1_ref.at[:, pl.ds(e_base, _GE)])
            pltpu.sync_copy(t1, g2_ref.at[:, pl.ds(e_base, _GE)])

        for b in range(_NB):
            start(b, b)

        def loop_body(it, carry):
            for b in range(_NB):
                g = it * _NB + b
                drain_store(g, b)
                start(g + _NB, b)
            return carry

        lax.fori_loop(0, main_iters, loop_body, 0)

        for b in range(_NB):
            drain_store(main_iters * _NB + b, b)

    return body(nodes_pad, e0, e1)


def _cross(a, b):
    ax, ay, az = a
    bx, by, bz = b
    return (ay * bz - az * by, az * bx - ax * bz, ax * by - ay * bx)


def _qrot(q, v):
    qv = q[:3]
    qw = q[3]
    t = _cross(qv, v)
    t = (2.0 * t[0], 2.0 * t[1], 2.0 * t[2])
    u = _cross(qv, t)
    return (v[0] + qw * t[0] + u[0],
            v[1] + qw * t[1] + u[1],
            v[2] + qw * t[2] + u[2])


def _qmul(q, r):
    qx, qy, qz, qw = q
    rx, ry, rz, rw = r
    cx, cy, cz = _cross((qx, qy, qz), (rx, ry, rz))
    return (qw * rx + rw * qx + cx,
            qw * ry + rw * qy + cy,
            qw * rz + rw * qz + cz,
            qw * rw - (qx * rx + qy * ry + qz * rz))


def _se3_inv(t, q):
    qi = (-q[0], -q[1], -q[2], q[3])
    ti = _qrot(qi, t)
    return (-ti[0], -ti[1], -ti[2]), qi


def _se3_mul(ta, qa, tb, qb):
    r = _qrot(qa, tb)
    return (ta[0] + r[0], ta[1] + r[1], ta[2] + r[2]), _qmul(qa, qb)


def _tc_body(p_ref, a_ref, b_ref, o_ref):
    tp = (p_ref[0], p_ref[1], p_ref[2])
    qp = (p_ref[3], p_ref[4], p_ref[5], p_ref[6])
    t1 = (a_ref[0], a_ref[1], a_ref[2])
    q1 = (a_ref[3], a_ref[4], a_ref[5], a_ref[6])
    t2 = (b_ref[0], b_ref[1], b_ref[2])
    q2 = (b_ref[3], b_ref[4], b_ref[5], b_ref[6])

    tip, qip = _se3_inv(tp, qp)
    ti1, qi1 = _se3_inv(t1, q1)
    tc, qc = _se3_mul(tip, qip, ti1, qi1)
    te, qe = _se3_mul(tc, qc, t2, q2)

    sign = jnp.where(qe[3] < 0.0, -1.0, 1.0)
    qv = (qe[0] * sign, qe[1] * sign, qe[2] * sign)
    qw = qe[3] * sign
    n = jnp.sqrt(qv[0] * qv[0] + qv[1] * qv[1] + qv[2] * qv[2])
    theta = 2.0 * jnp.arctan2(n, qw)
    small = n < 1e-6
    n_safe = jnp.where(small, 1.0, n)
    factor = jnp.where(small, 2.0 / jnp.maximum(qw, 1e-6), theta / n_safe)
    phi = (factor * qv[0], factor * qv[1], factor * qv[2])

    th = jnp.sqrt(phi[0] * phi[0] + phi[1] * phi[1] + phi[2] * phi[2])
    small_t = th < 1e-6
    ths = jnp.where(small_t, 1.0, th)
    c = jnp.where(
        small_t,
        1.0 / 12.0,
        (1.0 - ths * jnp.sin(ths) / (2.0 * (1.0 - jnp.cos(ths))))
        / (ths * ths),
    )
    pt = _cross(phi, te)
    ppt = _cross(phi, pt)
    o_ref[0] = te[0] - 0.5 * pt[0] + c * ppt[0]
    o_ref[1] = te[1] - 0.5 * pt[1] + c * ppt[1]
    o_ref[2] = te[2] - 0.5 * pt[2] + c * ppt[2]
    o_ref[3] = phi[0]
    o_ref[4] = phi[1]
    o_ref[5] = phi[2]


def _tc_compute(pt, g1t, g2t, rblk):
    rows = pt.shape[1]
    grid = (rows // rblk,)
    return pl.pallas_call(
        _tc_body,
        grid=grid,
        in_specs=[
            pl.BlockSpec((7, rblk, _LANE), lambda i: (0, i, 0)),
            pl.BlockSpec((8, rblk, _LANE), lambda i: (0, i, 0)),
            pl.BlockSpec((8, rblk, _LANE), lambda i: (0, i, 0)),
        ],
        out_specs=pl.BlockSpec((6, rblk, _LANE), lambda i: (0, i, 0)),
        out_shape=jax.ShapeDtypeStruct((6, rows, _LANE), jnp.float32),
    )(pt, g1t, g2t)


def kernel(nodes, edges, poses):
    n_edges = edges.shape[0]
    rows = n_edges // _LANE
    chunk = _NW * _G * _NB
    rows_pad = ((rows + chunk - 1) // chunk) * chunk

    nodes_pad = jnp.pad(nodes, ((0, 0), (0, 1)))
    et = edges.T.reshape(2, rows, _LANE)
    et = jnp.pad(et, ((0, 0), (0, rows_pad - rows), (0, 0)))

    n_pad = rows_pad * _LANE
    pt = jnp.pad(poses.T, ((0, 0), (0, n_pad - n_edges)))
    pt = pt.reshape(7, rows_pad, _LANE)

    split = (rows_pad // chunk // 2) * chunk
    outs = []
    for lo, hi in ((0, split), (split, rows_pad)):
        g1t, g2t = _sc_gather(nodes_pad, et[0, lo:hi], et[1, lo:hi])
        g1t = g1t.reshape(8, hi - lo, _LANE)
        g2t = g2t.reshape(8, hi - lo, _LANE)
        outs.append(_tc_compute(pt[:, lo:hi], g1t, g2t, 128))
    out_t = jnp.concatenate(outs, axis=1)
    return out_t.reshape(6, n_pad)[:, :n_edges].T
